# baseline (speedup 1.0000x reference)
import numpy as np

import jax
import jax.numpy as jnp
from jax import lax
from jax.experimental import pallas as pl
from jax.experimental.pallas import tpu as pltpu
from jax.experimental.pallas import tpu_sc as plsc

N, C, D = 4096, 200, 128
NT = N * C
NCORE, NSUB = 2, 16
NW = NCORE * NSUB
TPW = NT // NW
CH = 128
NCHUNK = TPW // CH
GB = 128
NGB = CH // GB
L = 16
DB = D // L
DW = D // 2
NSLOT = 4

_PERM = np.array([32 * g + 16 * h + k
                  for g in range(D // 32) for k in range(L) for h in (0, 1)])


def _body(gid_hbm, mod_hbm, expr_hbm, tt_hbm, wg_hbm, wmext_hbm, wex_hbm,
          out_hbm, *scr):
    names = ("gbuf", "obuf", "gid", "mod", "tt", "expr", "m0f", "em2", "mxb",
             "gsem", "osem", "ssem")
    slots = [dict(zip(names, scr[s * len(names):(s + 1) * len(names)]))
             for s in range(NSLOT)]
    wm_v, wex_v = scr[NSLOT * len(names):]

    wid = lax.axis_index("s") * NCORE + lax.axis_index("c")
    base0 = wid * TPW

    pltpu.sync_copy(wmext_hbm, wm_v)
    pltpu.sync_copy(wex_hbm, wex_v)
    wvecs = [wex_v[pl.ds(L * j, L)] for j in range(DB)]
    iota = lax.iota(jnp.int32, L)

    def chunk_base(i):
        return base0 + i * CH

    def scalar_copies(i, S):
        b = chunk_base(i)
        return [
            (gid_hbm.at[pl.ds(b, CH)], S["gid"]),
            (mod_hbm.at[pl.ds(b, CH)], S["mod"]),
            (tt_hbm.at[pl.ds(b, CH)], S["tt"]),
            (expr_hbm.at[pl.ds(b, CH)], S["expr"]),
        ]

    def fire_scalars(i, S):
        for src, dst in scalar_copies(i, S):
            pltpu.async_copy(src, dst, S["ssem"])

    def wait_scalars(i, S):
        for src, dst in scalar_copies(i, S):
            pltpu.make_async_copy(src, dst, S["ssem"]).wait()

    def precompute(S):
        for ii in range(CH // L):
            s = pl.ds(ii * L, L)
            tt = S["tt"][s]
            S["m0f"][s] = (tt & 1).astype(jnp.float32)
            S["em2"][s] = S["expr"][s] * ((tt >> 2) & 1).astype(jnp.float32)
            S["mxb"][s] = (((tt >> 1) & 1) << 10) | (S["mod"][s] << 7)

    def gather_copies(S):
        return [
            (wg_hbm.at[S["gid"].at[pl.ds(j * GB, GB)]],
             S["gbuf"].at[pl.ds(j * GB, GB)])
            for j in range(NGB)
        ]

    def fire_gathers(S):
        for src, dst in gather_copies(S):
            pltpu.async_copy(src, dst, S["gsem"])

    def wait_gathers(S):
        for src, dst in gather_copies(S):
            pltpu.make_async_copy(src, dst, S["gsem"]).wait()

    def token_pass(S):
        m0f, em2, mxb = S["m0f"], S["em2"], S["mxb"]
        gbuf, obuf = S["gbuf"], S["obuf"]

        @plsc.parallel_loop(0, CH, 1, unroll=8)
        def tok(t):
            vt = jnp.full((L,), t, jnp.int32)
            vm0 = plsc.load_gather(m0f, [vt])
            vem2 = plsc.load_gather(em2, [vt])
            mi = plsc.load_gather(mxb, [vt]) + iota
            for g in range(D // 32):
                vi = gbuf[t, pl.ds(g * L, L)]
                pair = (plsc.bitcast(vi << 16, jnp.float32),
                        plsc.bitcast(vi & jnp.int32(-65536), jnp.float32))
                for h in (0, 1):
                    j = 2 * g + h
                    vmod = plsc.load_gather(wm_v, [mi + (j * L)])
                    obuf[t, pl.ds(j * L, L)] = (
                        pair[h] * vm0 + vmod + wvecs[j] * vem2)

    def fire_out(i, S):
        pltpu.async_copy(S["obuf"], out_hbm.at[pl.ds(chunk_base(i), CH)],
                         S["osem"])

    def wait_out(i, S):
        pltpu.make_async_copy(S["obuf"], out_hbm.at[pl.ds(chunk_base(i), CH)],
                              S["osem"]).wait()

    for s in range(NSLOT):
        fire_scalars(s, slots[s])
    for s in range(NSLOT - 1):
        wait_scalars(s, slots[s])
        precompute(slots[s])
        fire_gathers(slots[s])

    def ring(k, _):
        for b in range(NSLOT):
            i = NSLOT * k + b
            S = slots[b]
            Sp = slots[(b + NSLOT - 1) % NSLOT]

            wait_gathers(S)

            @pl.when(i + NSLOT < NCHUNK)
            def _():
                fire_scalars(i + NSLOT, S)

            token_pass(S)
            fire_out(i, S)

            @pl.when(i > 0)
            def _():
                wait_out(i - 1, Sp)

            @pl.when(i + NSLOT - 1 < NCHUNK)
            def _():
                wait_scalars(i + NSLOT - 1, Sp)
                precompute(Sp)
                fire_gathers(Sp)
        return 0

    lax.fori_loop(0, NCHUNK // NSLOT, ring, 0)
    wait_out(NCHUNK - 1, slots[(NCHUNK - 1) % NSLOT])


_slot_scratch = [
    pltpu.VMEM((CH, DW), jnp.int32),
    pltpu.VMEM((CH, D), jnp.float32),
    pltpu.VMEM((CH,), jnp.int32),
    pltpu.VMEM((CH,), jnp.int32),
    pltpu.VMEM((CH,), jnp.int32),
    pltpu.VMEM((CH,), jnp.float32),
    pltpu.VMEM((CH,), jnp.float32),
    pltpu.VMEM((CH,), jnp.float32),
    pltpu.VMEM((CH,), jnp.int32),
    pltpu.SemaphoreType.DMA,
    pltpu.SemaphoreType.DMA,
    pltpu.SemaphoreType.DMA,
]

_sc_call = pl.kernel(
    _body,
    out_type=jax.ShapeDtypeStruct((NT, D), jnp.float32),
    mesh=plsc.VectorSubcoreMesh(core_axis_name="c", subcore_axis_name="s"),
    compiler_params=pltpu.CompilerParams(needs_layout_passes=False,
                                         use_tc_tiling_on_sc=False),
    scratch_types=(
        _slot_scratch * NSLOT
        + [
            pltpu.VMEM((16 * D,), jnp.float32),
            pltpu.VMEM((D,), jnp.float32),
        ]
    ),
)


@jax.jit
def kernel(gene_id, modality, expression, token_type_nc, W_gene, W_modality,
           w_expr):
    gid = gene_id.reshape(NT).astype(jnp.int32)
    mod = modality.reshape(NT).astype(jnp.int32)
    tt = token_type_nc.reshape(NT).astype(jnp.int32)
    expr = expression.reshape(NT).astype(jnp.float32)
    wg_packed = lax.bitcast_convert_type(
        W_gene.astype(jnp.bfloat16)[:, _PERM].reshape(-1, DW, 2), jnp.int32)
    wmext = jnp.concatenate(
        [jnp.zeros((8, D), jnp.float32), W_modality.astype(jnp.float32)],
        axis=0).reshape(-1)
    out = _sc_call(gid, mod, expr, tt, wg_packed, wmext, w_expr)
    return out.reshape(N, C, D)
